# baseline (speedup 1.0000x reference)
import functools

import jax
import jax.numpy as jnp
from jax import lax
from jax.experimental import pallas as pl
from jax.experimental.pallas import tpu as pltpu
from jax.experimental.pallas import tpu_sc as plsc

BATCH = 1024
TIME = 50
T48 = 48
D = 1000
DM = 896
DP = 1024
NC, NS = 2, 16
NW = NC * NS
B_PER_W = BATCH // NW
TAIL_OFF = (0, 16, 32, 48, 64, 80, 96)

_mesh = plsc.VectorSubcoreMesh(core_axis_name="c", subcore_axis_name="s")


@functools.partial(
    pl.kernel,
    mesh=_mesh,
    out_type=jax.ShapeDtypeStruct((BATCH, TIME, D), jnp.float32),
    scratch_types=[
        pltpu.VMEM((B_PER_W, T48), jnp.int32),
        pltpu.VMEM((B_PER_W, 8), jnp.int32),
        pltpu.VMEM((TIME, D), jnp.float32),
        pltpu.VMEM((TIME, D), jnp.float32),
        pltpu.VMEM((TIME, 128), jnp.float32),
        pltpu.SemaphoreType.DMA,
        pltpu.SemaphoreType.DMA,
        pltpu.SemaphoreType.DMA,
        pltpu.SemaphoreType.DMA,
        pltpu.SemaphoreType.DMA,
    ],
    compiler_params=pltpu.CompilerParams(use_tc_tiling_on_sc=True,
                                         disable_bounds_checks=True),
)
def _embed(idxa_hbm, idxb_hbm, table_hbm, out_hbm, idx_a, idx_b,
           buf0, buf1, tb, g0, g1, gt, w0, w1):
    wid = lax.axis_index("s") * NC + lax.axis_index("c")
    base = wid * B_PER_W
    dyn48 = pl.multiple_of(wid * 0 + T48, 8)

    pltpu.sync_copy(idxa_hbm.at[wid], idx_a)
    pltpu.sync_copy(idxb_hbm.at[wid], idx_b)

    main_src = table_hbm.at[:, pl.ds(0, DM)]
    tail_src = table_hbm.at[:, pl.ds(DM, 128)]

    def main_pieces(c, buf):
        yield main_src.at[idx_a.at[c]], buf.at[pl.ds(0, T48), pl.ds(0, DM)]
        yield main_src.at[idx_b.at[c]], buf.at[pl.ds(dyn48, 8), pl.ds(0, DM)]

    def tail_pieces(c):
        yield tail_src.at[idx_a.at[c]], tb.at[pl.ds(0, T48)]
        yield tail_src.at[idx_b.at[c]], tb.at[pl.ds(dyn48, 8)]

    def main_start(c, buf, sem):
        for src, dst in main_pieces(c, buf):
            pltpu.async_copy(src, dst, sem)

    def main_wait(c, buf, sem):
        for src, dst in main_pieces(c, buf):
            pltpu.make_async_copy(src, dst, sem).wait()

    def tail_start(c):
        for src, dst in tail_pieces(c):
            pltpu.async_copy(src, dst, gt)

    def tail_wait(c):
        for src, dst in tail_pieces(c):
            pltpu.make_async_copy(src, dst, gt).wait()

    def stitch_tail(buf):
        def row(t, carry):
            for off in TAIL_OFF[:-1]:
                buf[t, pl.ds(DM + off, 16)] = tb[t, pl.ds(off, 16)]
            off = TAIL_OFF[-1]
            start = pl.multiple_of(t * 0 + (DM + off), 16)
            buf[t, pl.ds(start, 16)] = tb[t, pl.ds(off, 16)]
            return carry
        lax.fori_loop(0, TIME, row, 0)

    def write_start(c, buf, sem):
        return pltpu.async_copy(buf, out_hbm.at[base + c], sem)

    def write_wait(c, buf, sem):
        pltpu.make_async_copy(buf, out_hbm.at[base + c], sem).wait()

    tail_start(0)
    main_start(0, buf0, g0)
    main_start(1, buf1, g1)

    def body(j, carry):
        c0 = 2 * j
        main_wait(c0, buf0, g0)
        tail_wait(c0)
        stitch_tail(buf0)
        tail_start(c0 + 1)
        write_start(c0, buf0, w0)
        main_wait(c0 + 1, buf1, g1)
        tail_wait(c0 + 1)
        stitch_tail(buf1)
        tail_start(c0 + 2)
        write_start(c0 + 1, buf1, w1)
        write_wait(c0, buf0, w0)
        main_start(c0 + 2, buf0, g0)
        write_wait(c0 + 1, buf1, w1)
        main_start(c0 + 3, buf1, g1)
        return carry

    lax.fori_loop(0, B_PER_W // 2 - 1, body, 0)

    cL = B_PER_W - 2
    main_wait(cL, buf0, g0)
    tail_wait(cL)
    stitch_tail(buf0)
    tail_start(cL + 1)
    hw0 = write_start(cL, buf0, w0)
    main_wait(cL + 1, buf1, g1)
    tail_wait(cL + 1)
    stitch_tail(buf1)
    hw1 = write_start(cL + 1, buf1, w1)
    hw0.wait()
    hw1.wait()


def kernel(idx, table):
    idx32 = idx.astype(jnp.int32)
    idx_a = idx32[:, :T48].reshape(NW, B_PER_W, T48)
    idx_b = jnp.concatenate([idx32[:, T48:], idx32[:, :8 - (TIME - T48)]],
                            axis=1).reshape(NW, B_PER_W, 8)
    table_p = jnp.pad(table, ((0, 0), (0, DP - D)))
    return _embed(idx_a, idx_b, table_p)


# speedup vs baseline: 1.1952x; 1.1952x over previous
import functools

import jax
import jax.numpy as jnp
from jax import lax
from jax.experimental import pallas as pl
from jax.experimental.pallas import tpu as pltpu
from jax.experimental.pallas import tpu_sc as plsc

BATCH = 1024
TIME = 50
TIME_P = 56
D = 1000
DP = 1024
NC, NS = 2, 16
NW = NC * NS
B_PER_W = BATCH // NW

_mesh = plsc.VectorSubcoreMesh(core_axis_name="c", subcore_axis_name="s")


@functools.partial(
    pl.kernel,
    mesh=_mesh,
    out_type=jax.ShapeDtypeStruct((BATCH, TIME_P, DP), jnp.float32),
    scratch_types=[
        pltpu.VMEM((B_PER_W, TIME_P), jnp.int32),
        pltpu.VMEM((TIME_P, DP), jnp.float32),
        pltpu.VMEM((TIME_P, DP), jnp.float32),
        pltpu.SemaphoreType.DMA,
        pltpu.SemaphoreType.DMA,
        pltpu.SemaphoreType.DMA,
        pltpu.SemaphoreType.DMA,
    ],
    compiler_params=pltpu.CompilerParams(use_tc_tiling_on_sc=True),
)
def _embed(idx_hbm, table_hbm, out_hbm, idx_v, buf0, buf1, g0, g1, w0, w1):
    wid = lax.axis_index("s") * NC + lax.axis_index("c")
    base = wid * B_PER_W

    pltpu.sync_copy(idx_hbm.at[wid], idx_v)

    def gather_start(c, buf, sem):
        return pltpu.async_copy(table_hbm.at[idx_v.at[c]], buf, sem)

    def gather_wait(c, buf, sem):
        pltpu.make_async_copy(table_hbm.at[idx_v.at[c]], buf, sem).wait()

    def write_start(c, buf, sem):
        return pltpu.async_copy(buf, out_hbm.at[base + c], sem)

    def write_wait(c, buf, sem):
        pltpu.make_async_copy(buf, out_hbm.at[base + c], sem).wait()

    gather_start(0, buf0, g0)
    gather_start(1, buf1, g1)

    def body(j, carry):
        c0 = 2 * j
        gather_wait(c0, buf0, g0)
        write_start(c0, buf0, w0)
        gather_wait(c0 + 1, buf1, g1)
        write_start(c0 + 1, buf1, w1)
        write_wait(c0, buf0, w0)
        gather_start(c0 + 2, buf0, g0)
        write_wait(c0 + 1, buf1, w1)
        gather_start(c0 + 3, buf1, g1)
        return carry

    lax.fori_loop(0, B_PER_W // 2 - 1, body, 0)

    cL = B_PER_W - 2
    gather_wait(cL, buf0, g0)
    hw0 = write_start(cL, buf0, w0)
    gather_wait(cL + 1, buf1, g1)
    hw1 = write_start(cL + 1, buf1, w1)
    hw0.wait()
    hw1.wait()


def kernel(idx, table):
    idx_p = jnp.pad(idx.astype(jnp.int32), ((0, 0), (0, TIME_P - TIME)),
                    mode="wrap")
    idx_r = idx_p.reshape(NW, B_PER_W, TIME_P)
    table_p = jnp.pad(table, ((0, 0), (0, DP - D)))
    out = _embed(idx_r, table_p)
    return out[:, :TIME, :D]
